# baseline (speedup 1.0000x reference)
import functools

import jax
import jax.numpy as jnp
from jax.experimental import pallas as pl
from jax.experimental.pallas import tpu as pltpu


_CH = 256
_SUB = 4
_DEPTH = 4


def _start_chunk(a_hbm, bufs, sems, c, slot):
    sub_rows = _CH // _SUB
    for s in range(_SUB):
        pltpu.make_async_copy(
            a_hbm.at[pl.ds(c * _CH + s * sub_rows, sub_rows), :],
            bufs.at[slot, pl.ds(s * sub_rows, sub_rows), :],
            sems.at[slot, s],
        ).start()


def _wait_chunk(a_hbm, bufs, sems, c, slot):
    sub_rows = _CH // _SUB
    for s in range(_SUB):
        pltpu.make_async_copy(
            a_hbm.at[pl.ds(c * _CH + s * sub_rows, sub_rows), :],
            bufs.at[slot, pl.ds(s * sub_rows, sub_rows), :],
            sems.at[slot, s],
        ).wait()


def _stream_mm_kernel(a_hbm, x_ref, o_ref, bufs, sems):
    n_chunks = a_hbm.shape[0] // _CH

    for s in range(_DEPTH):
        _start_chunk(a_hbm, bufs, sems, s, s)

    def body(c, carry):
        slot = jax.lax.rem(c, _DEPTH)
        _wait_chunk(a_hbm, bufs, sems, c, slot)
        o_ref[pl.ds(c * _CH, _CH), :] = jax.lax.dot_general(
            bufs[slot].astype(jnp.bfloat16), x_ref[...],
            (((1,), (0,)), ((), ())),
            preferred_element_type=jnp.float32,
        )
        nxt = c + _DEPTH

        @pl.when(nxt < n_chunks)
        def _():
            _start_chunk(a_hbm, bufs, sems, nxt, slot)

        return carry

    jax.lax.fori_loop(0, n_chunks, body, 0)


def _stream_mm_bias_kernel(a_hbm, x_ref, b_ref, o_ref, bufs, sems):
    n_chunks = a_hbm.shape[0] // _CH

    for s in range(_DEPTH):
        _start_chunk(a_hbm, bufs, sems, s, s)

    def body(c, carry):
        slot = jax.lax.rem(c, _DEPTH)
        _wait_chunk(a_hbm, bufs, sems, c, slot)
        rows = pl.ds(c * _CH, _CH)
        o_ref[rows, :] = b_ref[rows, :] + jax.lax.dot_general(
            bufs[slot].astype(jnp.bfloat16), x_ref[...],
            (((1,), (0,)), ((), ())),
            preferred_element_type=jnp.float32,
        )
        nxt = c + _DEPTH

        @pl.when(nxt < n_chunks)
        def _():
            _start_chunk(a_hbm, bufs, sems, nxt, slot)

        return carry

    jax.lax.fori_loop(0, n_chunks, body, 0)


@jax.jit
def _pass_mm(A, X):
    N, K = A.shape
    F = X.shape[1]
    return pl.pallas_call(
        _stream_mm_kernel,
        in_specs=[
            pl.BlockSpec(memory_space=pl.ANY),
            pl.BlockSpec(memory_space=pltpu.VMEM),
        ],
        out_specs=pl.BlockSpec(memory_space=pltpu.VMEM),
        out_shape=jax.ShapeDtypeStruct((N, F), jnp.float32),
        scratch_shapes=[
            pltpu.VMEM((_DEPTH, _CH, K), jnp.float32),
            pltpu.SemaphoreType.DMA((_DEPTH, _SUB)),
        ],
    )(A, X)


@jax.jit
def _pass_mm_bias(A, X, B):
    N, K = A.shape
    F = X.shape[1]
    return pl.pallas_call(
        _stream_mm_bias_kernel,
        in_specs=[
            pl.BlockSpec(memory_space=pl.ANY),
            pl.BlockSpec(memory_space=pltpu.VMEM),
            pl.BlockSpec(memory_space=pltpu.VMEM),
        ],
        out_specs=pl.BlockSpec(memory_space=pltpu.VMEM),
        out_shape=jax.ShapeDtypeStruct((N, F), jnp.float32),
        scratch_shapes=[
            pltpu.VMEM((_DEPTH, _CH, K), jnp.float32),
            pltpu.SemaphoreType.DMA((_DEPTH, _SUB)),
        ],
    )(A, X, B)


def kernel(A_p, A_n, x_p, x_n, w_p, w_n):
    D = x_p.shape[1]

    X1 = jnp.concatenate([x_p, x_n], axis=1).astype(jnp.bfloat16)
    Y1 = _pass_mm(A_p, X1)
    y1, t1 = Y1[:, :D], Y1[:, D:]

    X2 = jnp.concatenate([x_n, t1], axis=1).astype(jnp.bfloat16)
    Y2 = _pass_mm(A_n, X2)
    z1, t2 = Y2[:, :D], Y2[:, D:]

    X3 = jnp.concatenate(
        [w_p[2] * y1, w_n[1] * z1], axis=1).astype(jnp.bfloat16)
    PQ = jnp.concatenate(
        [w_p[0] * x_p + w_p[1] * y1, w_n[0] * z1 + w_n[2] * t2], axis=1)
    return _pass_mm_bias(A_p, X3, PQ)


# speedup vs baseline: 1.0228x; 1.0228x over previous
import functools

import jax
import jax.numpy as jnp
from jax.experimental import pallas as pl
from jax.experimental.pallas import tpu as pltpu


_CH = 256
_SUB = 1
_DEPTH = 3


def _start_chunk(a_hbm, bufs, sems, c, slot):
    sub_rows = _CH // _SUB
    for s in range(_SUB):
        pltpu.make_async_copy(
            a_hbm.at[pl.ds(c * _CH + s * sub_rows, sub_rows), :],
            bufs.at[slot, pl.ds(s * sub_rows, sub_rows), :],
            sems.at[slot, s],
        ).start()


def _wait_chunk(a_hbm, bufs, sems, c, slot):
    sub_rows = _CH // _SUB
    for s in range(_SUB):
        pltpu.make_async_copy(
            a_hbm.at[pl.ds(c * _CH + s * sub_rows, sub_rows), :],
            bufs.at[slot, pl.ds(s * sub_rows, sub_rows), :],
            sems.at[slot, s],
        ).wait()


def _stream_mm_kernel(a_hbm, x_ref, o_ref, bufs, sems):
    n_chunks = a_hbm.shape[0] // _CH

    for s in range(_DEPTH):
        _start_chunk(a_hbm, bufs, sems, s, s)

    def body(c, carry):
        slot = jax.lax.rem(c, _DEPTH)
        _wait_chunk(a_hbm, bufs, sems, c, slot)
        o_ref[pl.ds(c * _CH, _CH), :] = jax.lax.dot_general(
            bufs[slot].astype(jnp.bfloat16), x_ref[...],
            (((1,), (0,)), ((), ())),
            preferred_element_type=jnp.float32,
        )
        nxt = c + _DEPTH

        @pl.when(nxt < n_chunks)
        def _():
            _start_chunk(a_hbm, bufs, sems, nxt, slot)

        return carry

    jax.lax.fori_loop(0, n_chunks, body, 0)


def _stream_mm_bias_kernel(a_hbm, x_ref, b_ref, o_ref, bufs, sems):
    n_chunks = a_hbm.shape[0] // _CH

    for s in range(_DEPTH):
        _start_chunk(a_hbm, bufs, sems, s, s)

    def body(c, carry):
        slot = jax.lax.rem(c, _DEPTH)
        _wait_chunk(a_hbm, bufs, sems, c, slot)
        rows = pl.ds(c * _CH, _CH)
        o_ref[rows, :] = b_ref[rows, :] + jax.lax.dot_general(
            bufs[slot].astype(jnp.bfloat16), x_ref[...],
            (((1,), (0,)), ((), ())),
            preferred_element_type=jnp.float32,
        )
        nxt = c + _DEPTH

        @pl.when(nxt < n_chunks)
        def _():
            _start_chunk(a_hbm, bufs, sems, nxt, slot)

        return carry

    jax.lax.fori_loop(0, n_chunks, body, 0)


@jax.jit
def _pass_mm(A, X):
    N, K = A.shape
    F = X.shape[1]
    return pl.pallas_call(
        _stream_mm_kernel,
        in_specs=[
            pl.BlockSpec(memory_space=pl.ANY),
            pl.BlockSpec(memory_space=pltpu.VMEM),
        ],
        out_specs=pl.BlockSpec(memory_space=pltpu.VMEM),
        out_shape=jax.ShapeDtypeStruct((N, F), jnp.float32),
        scratch_shapes=[
            pltpu.VMEM((_DEPTH, _CH, K), jnp.float32),
            pltpu.SemaphoreType.DMA((_DEPTH, _SUB)),
        ],
    )(A, X)


@jax.jit
def _pass_mm_bias(A, X, B):
    N, K = A.shape
    F = X.shape[1]
    return pl.pallas_call(
        _stream_mm_bias_kernel,
        in_specs=[
            pl.BlockSpec(memory_space=pl.ANY),
            pl.BlockSpec(memory_space=pltpu.VMEM),
            pl.BlockSpec(memory_space=pltpu.VMEM),
        ],
        out_specs=pl.BlockSpec(memory_space=pltpu.VMEM),
        out_shape=jax.ShapeDtypeStruct((N, F), jnp.float32),
        scratch_shapes=[
            pltpu.VMEM((_DEPTH, _CH, K), jnp.float32),
            pltpu.SemaphoreType.DMA((_DEPTH, _SUB)),
        ],
    )(A, X, B)


def kernel(A_p, A_n, x_p, x_n, w_p, w_n):
    D = x_p.shape[1]

    X1 = jnp.concatenate([x_p, x_n], axis=1).astype(jnp.bfloat16)
    Y1 = _pass_mm(A_p, X1)
    y1, t1 = Y1[:, :D], Y1[:, D:]

    X2 = jnp.concatenate([x_n, t1], axis=1).astype(jnp.bfloat16)
    Y2 = _pass_mm(A_n, X2)
    z1, t2 = Y2[:, :D], Y2[:, D:]

    X3 = jnp.concatenate(
        [w_p[2] * y1, w_n[1] * z1], axis=1).astype(jnp.bfloat16)
    PQ = jnp.concatenate(
        [w_p[0] * x_p + w_p[1] * y1, w_n[0] * z1 + w_n[2] * t2], axis=1)
    return _pass_mm_bias(A_p, X3, PQ)
